# Initial kernel scaffold; baseline (speedup 1.0000x reference)
import jax
import jax.numpy as jnp
from jax.experimental import pallas as pl


def kernel(coords, bond_index, bond_rest_length, bond_force_constant, angle_index, angle_force_constant, angle_c0, angle_c1, angle_c2, angle_order, torsion_index, torsion_force_constant, torsion_order, torsion_cos_term, inversion_index, inversion_force_constant, inversion_c0, inversion_c1, inversion_c2, nonbond_index, vdw_minimum, vdw_well_depth, vdw_threshold):
    raise NotImplementedError("write your pallas kernel here")


# speedup vs baseline: 73.8711x; 73.8711x over previous
import functools

import jax
import jax.numpy as jnp
from jax import lax
from jax.experimental import pallas as pl
from jax.experimental.pallas import tpu as pltpu
from jax.experimental.pallas import tpu_sc as plsc

NC, NS, L = 2, 16, 16
NW = NC * NS

N_ATOMS = 10000
BOND_W = 5008
ANG_W = 5008
TOR_W = 5008
INV_W = 2512
NB_W = 20000
NB_H = 10000

_BUF = 10016

_f32 = jnp.float32
_i32 = jnp.int32


def _rsqrt(x):
    i = lax.bitcast_convert_type(x, _i32)
    i = jnp.int32(0x5F3759DF) - lax.shift_right_logical(i, 1)
    y = lax.bitcast_convert_type(i, _f32)
    y = y * (1.5 - 0.5 * x * y * y)
    y = y * (1.5 - 0.5 * x * y * y)
    return y


def _body(xs_h, ys_h, zs_h,
          b_i0, b_i1, b_r0, b_k,
          a_i0, a_i1, a_i2, a_od, a_k, a_c0, a_c1, a_c2,
          t_i0, t_i1, t_i2, t_i3, t_od, t_v, t_ct,
          v_i0, v_i1, v_i2, v_i3, v_k, v_c0, v_c1, v_c2,
          n_i0, n_i1, n_mn, n_dp, n_th,
          out_h,
          xs, ys, zs, ib0, ib1, ib2, ib3, ib4, pb0, pb1, pb2, pb3, ob):
    wid = lax.axis_index("s") * NC + lax.axis_index("c")

    pltpu.sync_copy(xs_h, xs)
    pltpu.sync_copy(ys_h, ys)
    pltpu.sync_copy(zs_h, zs)

    def stage(hbm, buf, base, n):
        pltpu.sync_copy(hbm.at[pl.ds(base, n)], buf.at[pl.ds(0, n)])

    def g3(idx):
        return (plsc.load_gather(xs, [idx]),
                plsc.load_gather(ys, [idx]),
                plsc.load_gather(zs, [idx]))

    acc0 = jnp.zeros((L,), _f32)

    base = wid * BOND_W
    stage(b_i0, ib0, base, BOND_W)
    stage(b_i1, ib1, base, BOND_W)
    stage(b_r0, pb0, base, BOND_W)
    stage(b_k, pb1, base, BOND_W)

    def bond_body(j, acc):
        sl = pl.ds(j * L, L)
        i0 = ib0[sl]
        i1 = ib1[sl]
        x0, y0, z0 = g3(i0)
        x1, y1, z1 = g3(i1)
        dx = x0 - x1
        dy = y0 - y1
        dz = z0 - z1
        d2 = dx * dx + dy * dy + dz * dz + 1e-12
        dist = d2 * _rsqrt(d2)
        diff = dist - pb0[sl]
        return acc + 0.5 * pb1[sl] * diff * diff

    acc = lax.fori_loop(0, BOND_W // L, bond_body, acc0)

    base = wid * ANG_W
    stage(a_i0, ib0, base, ANG_W)
    stage(a_i1, ib1, base, ANG_W)
    stage(a_i2, ib2, base, ANG_W)
    stage(a_od, ib3, base, ANG_W)
    stage(a_k, pb0, base, ANG_W)
    stage(a_c0, pb1, base, ANG_W)
    stage(a_c1, pb2, base, ANG_W)
    stage(a_c2, pb3, base, ANG_W)

    def angle_body(j, acc):
        sl = pl.ds(j * L, L)
        xi, yi, zi = g3(ib0[sl])
        xj, yj, zj = g3(ib1[sl])
        xk, yk, zk = g3(ib2[sl])
        v1x = xi - xj
        v1y = yi - yj
        v1z = zi - zj
        v2x = xk - xj
        v2y = yk - yj
        v2z = zk - zj
        dot = v1x * v2x + v1y * v2y + v1z * v2z
        s1 = v1x * v1x + v1y * v1y + v1z * v1z + 1e-12
        s2 = v2x * v2x + v2y * v2y + v2z * v2z + 1e-12
        c = dot * _rsqrt(s1 * s2)
        c = jnp.minimum(jnp.maximum(c, -1.0 + 1e-6), 1.0 - 1e-6)
        cc = c * c
        t2 = 2.0 * cc - 1.0
        kf = pb0[sl]
        e_gen = kf * (pb1[sl] + pb2[sl] * c + pb3[sl] * t2)
        od = ib3[sl]
        n = jnp.maximum(od, 1)
        t3 = c * (4.0 * cc - 3.0)
        t4 = 8.0 * cc * (cc - 1.0) + 1.0
        cn = jnp.where(n == 1, c, jnp.where(n == 2, t2, jnp.where(n == 3, t3, t4)))
        nf = n.astype(_f32)
        e_per = kf / (nf * nf) * (1.0 - cn)
        return acc + jnp.where(od == 0, e_gen, e_per)

    acc = lax.fori_loop(0, ANG_W // L, angle_body, acc)

    base = wid * TOR_W
    stage(t_i0, ib0, base, TOR_W)
    stage(t_i1, ib1, base, TOR_W)
    stage(t_i2, ib2, base, TOR_W)
    stage(t_i3, ib3, base, TOR_W)
    stage(t_od, ib4, base, TOR_W)
    stage(t_v, pb0, base, TOR_W)
    stage(t_ct, pb1, base, TOR_W)

    def torsion_body(j, acc):
        sl = pl.ds(j * L, L)
        x1, y1, z1 = g3(ib0[sl])
        x2, y2, z2 = g3(ib1[sl])
        x3, y3, z3 = g3(ib2[sl])
        x4, y4, z4 = g3(ib3[sl])
        b1x = x2 - x1
        b1y = y2 - y1
        b1z = z2 - z1
        b2x = x3 - x2
        b2y = y3 - y2
        b2z = z3 - z2
        b3x = x4 - x3
        b3y = y4 - y3
        b3z = z4 - z3
        n1x = b1y * b2z - b1z * b2y
        n1y = b1z * b2x - b1x * b2z
        n1z = b1x * b2y - b1y * b2x
        n2x = b2y * b3z - b2z * b3y
        n2y = b2z * b3x - b2x * b3z
        n2z = b2x * b3y - b2y * b3x
        mx = n1y * b2z - n1z * b2y
        my = n1z * b2x - n1x * b2z
        mz = n1x * b2y - n1y * b2x
        x = n1x * n2x + n1y * n2y + n1z * n2z
        t = mx * n2x + my * n2y + mz * n2z
        sb2 = b2x * b2x + b2y * b2y + b2z * b2z + 1e-12
        ym = t * _rsqrt(sb2)
        xp = x + 1e-12
        cphi = xp * _rsqrt(xp * xp + ym * ym + 1e-30)
        ccp = cphi * cphi
        tt2 = 2.0 * ccp - 1.0
        tt3 = cphi * (4.0 * ccp - 3.0)
        od = ib4[sl]
        cn = jnp.where(od == 1, cphi, jnp.where(od == 2, tt2, tt3))
        return acc + 0.5 * pb0[sl] * (1.0 - pb1[sl] * cn)

    acc = lax.fori_loop(0, TOR_W // L, torsion_body, acc)

    base = wid * INV_W
    stage(v_i0, ib0, base, INV_W)
    stage(v_i1, ib1, base, INV_W)
    stage(v_i2, ib2, base, INV_W)
    stage(v_i3, ib3, base, INV_W)
    stage(v_k, pb0, base, INV_W)
    stage(v_c0, pb1, base, INV_W)
    stage(v_c1, pb2, base, INV_W)
    stage(v_c2, pb3, base, INV_W)

    def inv_body(j, acc):
        sl = pl.ds(j * L, L)
        xi, yi, zi = g3(ib0[sl])
        xj, yj, zj = g3(ib1[sl])
        xk, yk, zk = g3(ib2[sl])
        xl, yl, zl = g3(ib3[sl])
        jx = xj - xi
        jy = yj - yi
        jz = zj - zi
        kx = xk - xi
        ky = yk - yi
        kz = zk - zi
        lx = xl - xi
        ly = yl - yi
        lz = zl - zi
        nx = jy * kz - jz * ky
        ny = jz * kx - jx * kz
        nz = jx * ky - jy * kx
        dot = nx * lx + ny * ly + nz * lz
        sn = nx * nx + ny * ny + nz * nz + 1e-12
        sls = lx * lx + ly * ly + lz * lz + 1e-12
        sy = dot * _rsqrt(sn * sls)
        sy = jnp.minimum(jnp.maximum(sy, -1.0 + 1e-6), 1.0 - 1e-6)
        c2w = 1.0 - 2.0 * sy * sy
        return acc + pb0[sl] * (pb1[sl] + pb2[sl] * sy + pb3[sl] * c2w)

    acc = lax.fori_loop(0, INV_W // L, inv_body, acc)

    def nb_body(j, acc):
        sl = pl.ds(j * L, L)
        i0 = ib0[sl]
        i1 = ib1[sl]
        x0, y0, z0 = g3(i0)
        x1, y1, z1 = g3(i1)
        dx = x0 - x1
        dy = y0 - y1
        dz = z0 - z1
        d2 = dx * dx + dy * dy + dz * dz + 1e-12
        mn = pb0[sl]
        q = (mn * mn) / d2
        x6 = q * q * q
        th = pb2[sl]
        e = pb1[sl] * (x6 * x6 - 2.0 * x6)
        return acc + jnp.where(d2 < th * th, e, 0.0)

    for half in range(2):
        base = wid * NB_W + half * NB_H
        stage(n_i0, ib0, base, NB_H)
        stage(n_i1, ib1, base, NB_H)
        stage(n_mn, pb0, base, NB_H)
        stage(n_dp, pb1, base, NB_H)
        stage(n_th, pb2, base, NB_H)
        acc = lax.fori_loop(0, NB_H // L, nb_body, acc)

    ob[...] = acc
    pltpu.sync_copy(ob, out_h.at[wid])


@functools.partial(
    pl.kernel,
    out_type=jax.ShapeDtypeStruct((NW, L), _f32),
    mesh=plsc.VectorSubcoreMesh(core_axis_name="c", subcore_axis_name="s",
                                num_cores=NC, num_subcores=NS),
    compiler_params=pltpu.CompilerParams(needs_layout_passes=False),
    scratch_types=[
        pltpu.VMEM((N_ATOMS,), _f32),
        pltpu.VMEM((N_ATOMS,), _f32),
        pltpu.VMEM((N_ATOMS,), _f32),
        pltpu.VMEM((_BUF,), _i32),
        pltpu.VMEM((_BUF,), _i32),
        pltpu.VMEM((_BUF,), _i32),
        pltpu.VMEM((_BUF,), _i32),
        pltpu.VMEM((_BUF,), _i32),
        pltpu.VMEM((_BUF,), _f32),
        pltpu.VMEM((_BUF,), _f32),
        pltpu.VMEM((_BUF,), _f32),
        pltpu.VMEM((_BUF,), _f32),
        pltpu.VMEM((L,), _f32),
    ],
)
def _uff_sc(*refs):
    _body(*refs)


def _pad(a, total):
    a = a.reshape(-1)
    n = a.shape[0]
    if n == total:
        return a
    return jnp.pad(a, (0, total - n))


def kernel(coords, bond_index, bond_rest_length, bond_force_constant,
           angle_index, angle_force_constant, angle_c0, angle_c1, angle_c2,
           angle_order, torsion_index, torsion_force_constant, torsion_order,
           torsion_cos_term, inversion_index, inversion_force_constant,
           inversion_c0, inversion_c1, inversion_c2, nonbond_index,
           vdw_minimum, vdw_well_depth, vdw_threshold):
    f = _f32
    i = _i32
    args = (
        coords[:, 0].astype(f), coords[:, 1].astype(f), coords[:, 2].astype(f),
        _pad(bond_index[:, 0].astype(i), NW * BOND_W),
        _pad(bond_index[:, 1].astype(i), NW * BOND_W),
        _pad(bond_rest_length.astype(f), NW * BOND_W),
        _pad(bond_force_constant.astype(f), NW * BOND_W),
        _pad(angle_index[:, 0].astype(i), NW * ANG_W),
        _pad(angle_index[:, 1].astype(i), NW * ANG_W),
        _pad(angle_index[:, 2].astype(i), NW * ANG_W),
        _pad(angle_order.astype(i), NW * ANG_W),
        _pad(angle_force_constant.astype(f), NW * ANG_W),
        _pad(angle_c0.astype(f), NW * ANG_W),
        _pad(angle_c1.astype(f), NW * ANG_W),
        _pad(angle_c2.astype(f), NW * ANG_W),
        _pad(torsion_index[:, 0].astype(i), NW * TOR_W),
        _pad(torsion_index[:, 1].astype(i), NW * TOR_W),
        _pad(torsion_index[:, 2].astype(i), NW * TOR_W),
        _pad(torsion_index[:, 3].astype(i), NW * TOR_W),
        _pad(torsion_order.astype(i), NW * TOR_W),
        _pad(torsion_force_constant.astype(f), NW * TOR_W),
        _pad(torsion_cos_term.astype(f), NW * TOR_W),
        _pad(inversion_index[:, 0].astype(i), NW * INV_W),
        _pad(inversion_index[:, 1].astype(i), NW * INV_W),
        _pad(inversion_index[:, 2].astype(i), NW * INV_W),
        _pad(inversion_index[:, 3].astype(i), NW * INV_W),
        _pad(inversion_force_constant.astype(f), NW * INV_W),
        _pad(inversion_c0.astype(f), NW * INV_W),
        _pad(inversion_c1.astype(f), NW * INV_W),
        _pad(inversion_c2.astype(f), NW * INV_W),
        _pad(nonbond_index[:, 0].astype(i), NW * NB_W),
        _pad(nonbond_index[:, 1].astype(i), NW * NB_W),
        _pad(vdw_minimum.astype(f), NW * NB_W),
        _pad(vdw_well_depth.astype(f), NW * NB_W),
        _pad(vdw_threshold.astype(f), NW * NB_W),
    )
    partials = _uff_sc(*args)
    return jnp.sum(partials)


# speedup vs baseline: 81.0330x; 1.0970x over previous
import functools

import jax
import jax.numpy as jnp
from jax import lax
from jax.experimental import pallas as pl
from jax.experimental.pallas import tpu as pltpu
from jax.experimental.pallas import tpu_sc as plsc

NC, NS, L = 2, 16, 16
NW = NC * NS

N_ATOMS = 10000
BOND_W = 5120
ANG_W = 5120
TOR_W = 5120
INV_W = 2560
NB_W = 20480
NB_H = 10240

_BUF = 10240

_f32 = jnp.float32
_i32 = jnp.int32


def _rsqrt(x):
    i = lax.bitcast_convert_type(x, _i32)
    i = jnp.int32(0x5F3759DF) - lax.shift_right_logical(i, 1)
    y = lax.bitcast_convert_type(i, _f32)
    y = y * (1.5 - 0.5 * x * y * y)
    y = y * (1.5 - 0.5 * x * y * y)
    return y


def _body(xs_h, ys_h, zs_h,
          b_i0, b_i1, b_r0, b_k,
          a_i0, a_i1, a_i2, a_od, a_k, a_c0, a_c1, a_c2,
          t_i0, t_i1, t_i2, t_i3, t_od, t_v, t_ct,
          v_i0, v_i1, v_i2, v_i3, v_k, v_c0, v_c1, v_c2,
          n_i0, n_i1, n_mn, n_dp, n_th,
          out_h,
          xs, ys, zs, ib0, ib1, ib2, ib3, ib4, pb0, pb1, pb2, pb3, ob, sem):
    wid = lax.axis_index("s") * NC + lax.axis_index("c")

    pltpu.sync_copy(xs_h, xs)
    pltpu.sync_copy(ys_h, ys)
    pltpu.sync_copy(zs_h, zs)

    def stage_all(base, n, pairs):
        cps = [pltpu.async_copy(hbm.at[pl.ds(base, n)], buf.at[pl.ds(0, n)],
                                sem)
               for hbm, buf in pairs]
        for c in cps:
            c.wait()

    def g3(idx):
        return (plsc.load_gather(xs, [idx]),
                plsc.load_gather(ys, [idx]),
                plsc.load_gather(zs, [idx]))

    def unrolled(n_vregs, unroll, body_fn, acc):
        @plsc.parallel_loop(0, n_vregs, unroll=unroll, carry=acc)
        def final(j, a):
            return a + body_fn(j)
        return final

    acc0 = jnp.zeros((L,), _f32)

    stage_all(wid * BOND_W, BOND_W,
              [(b_i0, ib0), (b_i1, ib1), (b_r0, pb0), (b_k, pb1)])

    def bond_body(j):
        sl = pl.ds(j * L, L)
        i0 = ib0[sl]
        i1 = ib1[sl]
        x0, y0, z0 = g3(i0)
        x1, y1, z1 = g3(i1)
        dx = x0 - x1
        dy = y0 - y1
        dz = z0 - z1
        d2 = dx * dx + dy * dy + dz * dz + 1e-12
        dist = d2 * _rsqrt(d2)
        diff = dist - pb0[sl]
        return 0.5 * pb1[sl] * diff * diff

    acc = unrolled(BOND_W // L, 4, bond_body, acc0)

    stage_all(wid * ANG_W, ANG_W,
              [(a_i0, ib0), (a_i1, ib1), (a_i2, ib2), (a_od, ib3),
               (a_k, pb0), (a_c0, pb1), (a_c1, pb2), (a_c2, pb3)])

    def angle_body(j):
        sl = pl.ds(j * L, L)
        xi, yi, zi = g3(ib0[sl])
        xj, yj, zj = g3(ib1[sl])
        xk, yk, zk = g3(ib2[sl])
        v1x = xi - xj
        v1y = yi - yj
        v1z = zi - zj
        v2x = xk - xj
        v2y = yk - yj
        v2z = zk - zj
        dot = v1x * v2x + v1y * v2y + v1z * v2z
        s1 = v1x * v1x + v1y * v1y + v1z * v1z + 1e-12
        s2 = v2x * v2x + v2y * v2y + v2z * v2z + 1e-12
        c = dot * _rsqrt(s1 * s2)
        c = jnp.minimum(jnp.maximum(c, -1.0 + 1e-6), 1.0 - 1e-6)
        cc = c * c
        t2 = 2.0 * cc - 1.0
        kf = pb0[sl]
        e_gen = kf * (pb1[sl] + pb2[sl] * c + pb3[sl] * t2)
        od = ib3[sl]
        n = jnp.maximum(od, 1)
        t3 = c * (4.0 * cc - 3.0)
        t4 = 8.0 * cc * (cc - 1.0) + 1.0
        cn = jnp.where(n == 1, c, jnp.where(n == 2, t2, jnp.where(n == 3, t3, t4)))
        nf = n.astype(_f32)
        e_per = kf / (nf * nf) * (1.0 - cn)
        return jnp.where(od == 0, e_gen, e_per)

    acc = unrolled(ANG_W // L, 2, angle_body, acc)

    stage_all(wid * TOR_W, TOR_W,
              [(t_i0, ib0), (t_i1, ib1), (t_i2, ib2), (t_i3, ib3),
               (t_od, ib4), (t_v, pb0), (t_ct, pb1)])

    def torsion_body(j):
        sl = pl.ds(j * L, L)
        x1, y1, z1 = g3(ib0[sl])
        x2, y2, z2 = g3(ib1[sl])
        x3, y3, z3 = g3(ib2[sl])
        x4, y4, z4 = g3(ib3[sl])
        b1x = x2 - x1
        b1y = y2 - y1
        b1z = z2 - z1
        b2x = x3 - x2
        b2y = y3 - y2
        b2z = z3 - z2
        b3x = x4 - x3
        b3y = y4 - y3
        b3z = z4 - z3
        n1x = b1y * b2z - b1z * b2y
        n1y = b1z * b2x - b1x * b2z
        n1z = b1x * b2y - b1y * b2x
        n2x = b2y * b3z - b2z * b3y
        n2y = b2z * b3x - b2x * b3z
        n2z = b2x * b3y - b2y * b3x
        mx = n1y * b2z - n1z * b2y
        my = n1z * b2x - n1x * b2z
        mz = n1x * b2y - n1y * b2x
        x = n1x * n2x + n1y * n2y + n1z * n2z
        t = mx * n2x + my * n2y + mz * n2z
        sb2 = b2x * b2x + b2y * b2y + b2z * b2z + 1e-12
        ym = t * _rsqrt(sb2)
        xp = x + 1e-12
        cphi = xp * _rsqrt(xp * xp + ym * ym + 1e-30)
        ccp = cphi * cphi
        tt2 = 2.0 * ccp - 1.0
        tt3 = cphi * (4.0 * ccp - 3.0)
        od = ib4[sl]
        cn = jnp.where(od == 1, cphi, jnp.where(od == 2, tt2, tt3))
        return 0.5 * pb0[sl] * (1.0 - pb1[sl] * cn)

    acc = unrolled(TOR_W // L, 2, torsion_body, acc)

    stage_all(wid * INV_W, INV_W,
              [(v_i0, ib0), (v_i1, ib1), (v_i2, ib2), (v_i3, ib3),
               (v_k, pb0), (v_c0, pb1), (v_c1, pb2), (v_c2, pb3)])

    def inv_body(j):
        sl = pl.ds(j * L, L)
        xi, yi, zi = g3(ib0[sl])
        xj, yj, zj = g3(ib1[sl])
        xk, yk, zk = g3(ib2[sl])
        xl, yl, zl = g3(ib3[sl])
        jx = xj - xi
        jy = yj - yi
        jz = zj - zi
        kx = xk - xi
        ky = yk - yi
        kz = zk - zi
        lx = xl - xi
        ly = yl - yi
        lz = zl - zi
        nx = jy * kz - jz * ky
        ny = jz * kx - jx * kz
        nz = jx * ky - jy * kx
        dot = nx * lx + ny * ly + nz * lz
        sn = nx * nx + ny * ny + nz * nz + 1e-12
        sls = lx * lx + ly * ly + lz * lz + 1e-12
        sy = dot * _rsqrt(sn * sls)
        sy = jnp.minimum(jnp.maximum(sy, -1.0 + 1e-6), 1.0 - 1e-6)
        c2w = 1.0 - 2.0 * sy * sy
        return pb0[sl] * (pb1[sl] + pb2[sl] * sy + pb3[sl] * c2w)

    acc = unrolled(INV_W // L, 2, inv_body, acc)

    def nb_body(j):
        sl = pl.ds(j * L, L)
        i0 = ib0[sl]
        i1 = ib1[sl]
        x0, y0, z0 = g3(i0)
        x1, y1, z1 = g3(i1)
        dx = x0 - x1
        dy = y0 - y1
        dz = z0 - z1
        d2 = dx * dx + dy * dy + dz * dz + 1e-12
        mn = pb0[sl]
        q = (mn * mn) / d2
        x6 = q * q * q
        th = pb2[sl]
        e = pb1[sl] * (x6 * x6 - 2.0 * x6)
        return jnp.where(d2 < th * th, e, 0.0)

    for half in range(2):
        stage_all(wid * NB_W + half * NB_H, NB_H,
                  [(n_i0, ib0), (n_i1, ib1), (n_mn, pb0), (n_dp, pb1),
                   (n_th, pb2)])
        acc = unrolled(NB_H // L, 4, nb_body, acc)

    ob[...] = acc
    pltpu.sync_copy(ob, out_h.at[wid])


@functools.partial(
    pl.kernel,
    out_type=jax.ShapeDtypeStruct((NW, L), _f32),
    mesh=plsc.VectorSubcoreMesh(core_axis_name="c", subcore_axis_name="s",
                                num_cores=NC, num_subcores=NS),
    compiler_params=pltpu.CompilerParams(needs_layout_passes=False),
    scratch_types=[
        pltpu.VMEM((N_ATOMS,), _f32),
        pltpu.VMEM((N_ATOMS,), _f32),
        pltpu.VMEM((N_ATOMS,), _f32),
        pltpu.VMEM((_BUF,), _i32),
        pltpu.VMEM((_BUF,), _i32),
        pltpu.VMEM((_BUF,), _i32),
        pltpu.VMEM((_BUF,), _i32),
        pltpu.VMEM((_BUF,), _i32),
        pltpu.VMEM((_BUF,), _f32),
        pltpu.VMEM((_BUF,), _f32),
        pltpu.VMEM((_BUF,), _f32),
        pltpu.VMEM((_BUF,), _f32),
        pltpu.VMEM((L,), _f32),
        pltpu.SemaphoreType.DMA,
    ],
)
def _uff_sc(*refs):
    _body(*refs)


def _pad(a, total):
    a = a.reshape(-1)
    n = a.shape[0]
    if n == total:
        return a
    return jnp.pad(a, (0, total - n))


def kernel(coords, bond_index, bond_rest_length, bond_force_constant,
           angle_index, angle_force_constant, angle_c0, angle_c1, angle_c2,
           angle_order, torsion_index, torsion_force_constant, torsion_order,
           torsion_cos_term, inversion_index, inversion_force_constant,
           inversion_c0, inversion_c1, inversion_c2, nonbond_index,
           vdw_minimum, vdw_well_depth, vdw_threshold):
    f = _f32
    i = _i32
    args = (
        coords[:, 0].astype(f), coords[:, 1].astype(f), coords[:, 2].astype(f),
        _pad(bond_index[:, 0].astype(i), NW * BOND_W),
        _pad(bond_index[:, 1].astype(i), NW * BOND_W),
        _pad(bond_rest_length.astype(f), NW * BOND_W),
        _pad(bond_force_constant.astype(f), NW * BOND_W),
        _pad(angle_index[:, 0].astype(i), NW * ANG_W),
        _pad(angle_index[:, 1].astype(i), NW * ANG_W),
        _pad(angle_index[:, 2].astype(i), NW * ANG_W),
        _pad(angle_order.astype(i), NW * ANG_W),
        _pad(angle_force_constant.astype(f), NW * ANG_W),
        _pad(angle_c0.astype(f), NW * ANG_W),
        _pad(angle_c1.astype(f), NW * ANG_W),
        _pad(angle_c2.astype(f), NW * ANG_W),
        _pad(torsion_index[:, 0].astype(i), NW * TOR_W),
        _pad(torsion_index[:, 1].astype(i), NW * TOR_W),
        _pad(torsion_index[:, 2].astype(i), NW * TOR_W),
        _pad(torsion_index[:, 3].astype(i), NW * TOR_W),
        _pad(torsion_order.astype(i), NW * TOR_W),
        _pad(torsion_force_constant.astype(f), NW * TOR_W),
        _pad(torsion_cos_term.astype(f), NW * TOR_W),
        _pad(inversion_index[:, 0].astype(i), NW * INV_W),
        _pad(inversion_index[:, 1].astype(i), NW * INV_W),
        _pad(inversion_index[:, 2].astype(i), NW * INV_W),
        _pad(inversion_index[:, 3].astype(i), NW * INV_W),
        _pad(inversion_force_constant.astype(f), NW * INV_W),
        _pad(inversion_c0.astype(f), NW * INV_W),
        _pad(inversion_c1.astype(f), NW * INV_W),
        _pad(inversion_c2.astype(f), NW * INV_W),
        _pad(nonbond_index[:, 0].astype(i), NW * NB_W),
        _pad(nonbond_index[:, 1].astype(i), NW * NB_W),
        _pad(vdw_minimum.astype(f), NW * NB_W),
        _pad(vdw_well_depth.astype(f), NW * NB_W),
        _pad(vdw_threshold.astype(f), NW * NB_W),
    )
    partials = _uff_sc(*args)
    return jnp.sum(partials)
